# baseline (speedup 1.0000x reference)
import functools

import jax
import jax.numpy as jnp
from jax import lax
from jax.experimental import pallas as pl
from jax.experimental.pallas import tpu as pltpu
from jax.experimental.pallas import tpu_sc as plsc

NUM_CORES = 2
NUM_SUBCORES = 16
NUM_WORKERS = NUM_CORES * NUM_SUBCORES
LANES = 16

B = 4096
T = 20
F = 26
V = 100000
D = 32
BL = 128
TG = T // 2


def _transpose_block(rows, tr, i, iota16, dsplats):
    base_i = i * BL
    for b16 in range(BL // LANES):
        rowv = iota16 + (base_i + b16 * LANES)
        for d in range(D):
            val = plsc.load_gather(rows, [rowv, dsplats[d]])
            tr[d // 8, d % 8, pl.ds(b16 * LANES, LANES)] = val


@jax.jit
def _gather_native(tab, idx_t):
    mesh = plsc.VectorSubcoreMesh(core_axis_name="c", subcore_axis_name="s")

    @functools.partial(
        pl.kernel,
        out_type=jax.ShapeDtypeStruct((T, F, D // 8, B // BL, 8, BL),
                                      jnp.float32),
        mesh=mesh,
        scratch_types=[
            pltpu.VMEM((T, BL), jnp.int32),
            pltpu.VMEM((TG * BL, D), jnp.float32),
            pltpu.VMEM((TG * BL, D), jnp.float32),
            pltpu.VMEM((D // 8, 8, BL), jnp.float32),
            pltpu.VMEM((D // 8, 8, BL), jnp.float32),
            pltpu.SemaphoreType.DMA,
            pltpu.SemaphoreType.DMA,
            pltpu.SemaphoreType.DMA,
            pltpu.SemaphoreType.DMA,
        ],
        compiler_params=pltpu.CompilerParams(
            use_tc_tiling_on_sc=False, needs_layout_passes=False
        ),
    )
    def k(tab_hbm, idx_hbm, out_hbm,
          idx_v, rows0, rows1, tr0, tr1,
          gsem0, gsem1, wsem0, wsem1):
        wid = lax.axis_index("s") * NUM_CORES + lax.axis_index("c")
        iota16 = lax.broadcasted_iota(jnp.int32, (LANES,), 0)
        dsplats = [jnp.full((LANES,), d, jnp.int32) for d in range(D)]

        def wait_tr(tr, wsem):
            pltpu.make_async_copy(
                tr, out_hbm.at[0, 0, :, wid, :, :], wsem
            ).wait()

        def item(f, tg, i, rows, tr, wsem, first):
            t = tg * TG + i
            if not first:
                wait_tr(tr, wsem)
            _transpose_block(rows, tr, i, iota16, dsplats)
            pltpu.async_copy(tr, out_hbm.at[t, f, :, wid, :, :], wsem)

        def body(f, carry):
            pltpu.sync_copy(idx_hbm.at[f, :, pl.ds(BL * wid, BL)], idx_v)
            fv = jnp.full((LANES,), f * V, jnp.int32)
            for r in range(T):
                for s in range(BL // LANES):
                    sl = pl.ds(LANES * s, LANES)
                    idx_v[r, sl] = idx_v[r, sl] + fv
            d0 = [
                pltpu.async_copy(tab_hbm.at[idx_v.at[j]],
                                 rows0.at[pl.ds(j * BL, BL)], gsem0)
                for j in range(TG)
            ]
            d1 = [
                pltpu.async_copy(tab_hbm.at[idx_v.at[TG + j]],
                                 rows1.at[pl.ds(j * BL, BL)], gsem1)
                for j in range(TG)
            ]
            for dsc in d0:
                dsc.wait()

            def inner0(ip, c):
                item(f, 0, 2 * ip, rows0, tr0, wsem0, False)
                item(f, 0, 2 * ip + 1, rows0, tr1, wsem1, False)
                return c

            def inner1(ip, c):
                item(f, 1, 2 * ip, rows1, tr0, wsem0, False)
                item(f, 1, 2 * ip + 1, rows1, tr1, wsem1, False)
                return c

            lax.fori_loop(0, TG // 2, inner0, 0)
            for dsc in d1:
                dsc.wait()
            lax.fori_loop(0, TG // 2, inner1, 0)
            return carry

        pltpu.async_copy(tr0, out_hbm.at[0, 0, :, wid, :, :], wsem0)
        pltpu.async_copy(tr1, out_hbm.at[0, 0, :, wid, :, :], wsem1)
        lax.fori_loop(0, F, body, 0)
        wait_tr(tr0, wsem0)
        wait_tr(tr1, wsem1)

    return k(tab, idx_t)


def kernel(input, tables):
    idx_t = jnp.transpose(input, (2, 1, 0))
    tab = tables.reshape(F * V, D)
    out6 = _gather_native(tab, idx_t)
    return out6.transpose(3, 5, 0, 1, 2, 4).reshape(B, T, F, D)


# speedup vs baseline: 1.1358x; 1.1358x over previous
import functools

import jax
import jax.numpy as jnp
from jax import lax
from jax.experimental import pallas as pl
from jax.experimental.pallas import tpu as pltpu
from jax.experimental.pallas import tpu_sc as plsc

NUM_CORES = 2
NUM_SUBCORES = 16
NUM_WORKERS = NUM_CORES * NUM_SUBCORES
LANES = 16

IDX_W = 128
CHUNK = 1664
GRP = CHUNK // IDX_W
PERIOD = 208


@functools.partial(jax.jit, static_argnums=(3, 4))
def _gather_flat(tab, idx3, off, n_rows, d):
    per_w = n_rows // NUM_WORKERS
    n_chunks = per_w // CHUNK
    n_chunks_total = n_rows // CHUNK
    mesh = plsc.VectorSubcoreMesh(core_axis_name="c", subcore_axis_name="s")

    @functools.partial(
        pl.kernel,
        out_type=jax.ShapeDtypeStruct((n_rows, d), jnp.float32),
        mesh=mesh,
        scratch_types=[
            pltpu.VMEM((GRP, IDX_W), jnp.int32),
            pltpu.VMEM((GRP, IDX_W), jnp.int32),
            pltpu.VMEM((CHUNK, d), jnp.float32),
            pltpu.VMEM((CHUNK, d), jnp.float32),
            pltpu.VMEM((PERIOD,), jnp.int32),
            pltpu.SemaphoreType.DMA,
            pltpu.SemaphoreType.DMA,
            pltpu.SemaphoreType.DMA,
            pltpu.SemaphoreType.DMA,
        ],
        compiler_params=pltpu.CompilerParams(use_tc_tiling_on_sc=False),
    )
    def k(tab_hbm, idx_hbm, off_hbm, out_hbm,
          idx_a, idx_b, rows_a, rows_b, off_v,
          gsem_a, gsem_b, wsem_a, wsem_b):
        wid = lax.axis_index("s") * NUM_CORES + lax.axis_index("c")
        idx_bufs = (idx_a, idx_b)
        row_bufs = (rows_a, rows_b)
        gsems = (gsem_a, gsem_b)
        wsems = (wsem_a, wsem_b)
        pltpu.sync_copy(off_hbm, off_v)

        def load_idx(g, buf):
            c = jnp.minimum(wid * n_chunks + g, n_chunks_total - 1)
            pltpu.sync_copy(idx_hbm.at[c], buf)
            for j in range(GRP):
                for t in range(IDX_W // LANES):
                    st = (j * IDX_W + t * LANES) % PERIOD
                    sl = pl.ds(t * LANES, LANES)
                    buf[j, sl] = buf[j, sl] + off_v[pl.ds(st, LANES)]

        def fire_gathers(ibuf, rbuf, sem):
            return [
                pltpu.async_copy(
                    tab_hbm.at[ibuf.at[j]],
                    rbuf.at[pl.ds(j * IDX_W, IDX_W)],
                    sem,
                )
                for j in range(GRP)
            ]

        def drain_and_write(g, rbuf, sem, descs, wsem):
            for dsc in descs:
                dsc.wait()
            return pltpu.async_copy(
                rbuf, out_hbm.at[pl.ds(wid * per_w + g * CHUNK, CHUNK)], wsem
            )

        def wait_write(b):
            pltpu.make_async_copy(
                row_bufs[b], out_hbm.at[pl.ds(0, CHUNK)], wsems[b]
            ).wait()

        load_idx(0, idx_bufs[0])
        for g in (0, 1):
            b = g % 2
            descs = fire_gathers(idx_bufs[b], row_bufs[b], gsems[b])
            load_idx(g + 1, idx_bufs[1 - b])
            drain_and_write(g, row_bufs[b], gsems[b], descs, wsems[b])

        def body(go, carry):
            for b in (0, 1):
                g = 2 * go + b
                wait_write(b)
                descs = fire_gathers(idx_bufs[b], row_bufs[b], gsems[b])
                load_idx(g + 1, idx_bufs[1 - b])
                drain_and_write(g, row_bufs[b], gsems[b], descs, wsems[b])
            return carry

        lax.fori_loop(1, n_chunks // 2, body, 0)
        wait_write(0)
        wait_write(1)

    return k(tab, idx3, off)


def kernel(input, tables):
    b, t, f = input.shape
    vocab, d = tables.shape[1], tables.shape[2]
    n_rows = b * t * f
    idx3 = input.reshape(n_rows // CHUNK, GRP, IDX_W)
    tab = tables.reshape(f * vocab, d)
    off = jnp.tile(jnp.arange(f, dtype=jnp.int32) * vocab, PERIOD // f)
    out = _gather_flat(tab, idx3, off, n_rows, d)
    return out.reshape(b, t, f, d)
